# baseline (speedup 1.0000x reference)
import functools

import jax
import jax.numpy as jnp
from jax import lax
from jax.experimental import pallas as pl
from jax.experimental.pallas import tpu as pltpu
from jax.experimental.pallas import tpu_sc as plsc

N = 10000
E = 320000
EPS = 1e-5

_NC, _NS, _L = 2, 16, 16
_NW = _NC * _NS
_CH = 80
_EPW = _CH * 128
_EPAD = _NW * _EPW
_NP = 10240
_D = 128
_BLK = 8


def _sc_deg_body(col_hbm, ew_hbm, out_hbm, idx_v, val_v, zb, sh):
    cid = lax.axis_index("c")
    sid = lax.axis_index("s")
    w = cid * _NS + sid

    @pl.loop(0, 40)
    def _zero(i):
        zb[pl.ds(i * 16, 16)] = jnp.zeros((16,), jnp.float32)

    pltpu.sync_copy(zb, sh.at[pl.ds(sid * 640, 640)])
    pltpu.sync_copy(col_hbm.at[w], idx_v)
    pltpu.sync_copy(ew_hbm.at[w], val_v)
    plsc.subcore_barrier()

    @pl.loop(0, _CH)
    def _scat(j):
        pltpu.sync_copy(val_v.at[j], sh.at[idx_v.at[j]], add=True)

    plsc.subcore_barrier()

    @pl.when(sid == 0)
    def _out():
        pltpu.sync_copy(sh, out_hbm.at[cid])


def _sc_deg(col3, ew3):
    return pl.kernel(
        _sc_deg_body,
        out_type=jax.ShapeDtypeStruct((_NC, _NP), jnp.float32),
        mesh=plsc.VectorSubcoreMesh(core_axis_name="c", subcore_axis_name="s"),
        scratch_types=[
            pltpu.VMEM((_CH, 128), jnp.int32),
            pltpu.VMEM((_CH, 128), jnp.float32),
            pltpu.VMEM((640,), jnp.float32),
            pltpu.VMEM_SHARED((_NP,), jnp.float32),
        ],
    )(col3, ew3)


def _splat(v16, e):
    idx = jnp.full((16, 1), e, jnp.int32)
    return lax.gather(
        v16, idx,
        lax.GatherDimensionNumbers(
            offset_dims=(), collapsed_slice_dims=(0,), start_index_map=(0,)),
        slice_sizes=(1,),
        mode=lax.GatherScatterMode.PROMISE_IN_BOUNDS,
    )


def _sc_scatter_body(src, row3, col3, ew3, out_hbm, rowb, colb, ewb, R2, sh, semg, sems):
    cid = lax.axis_index("c")
    sid = lax.axis_index("s")
    w = cid * _NS + sid

    @pl.loop(0, 128)
    def _z(r):
        for f in range(8):
            R2[0, r, pl.ds(f * 16, 16)] = jnp.zeros((16,), jnp.float32)

    for k in range(5):
        pltpu.sync_copy(R2.at[0], sh.at[pl.ds(sid * 640 + k * 128, 128)])

    plsc.subcore_barrier()

    pltpu.sync_copy(row3.at[w, pl.ds(0, _BLK)], rowb.at[0])
    pltpu.sync_copy(col3.at[w, pl.ds(0, _BLK)], colb.at[0])
    pltpu.sync_copy(ew3.at[w, pl.ds(0, _BLK)], ewb.at[0])
    pltpu.async_copy(src.at[rowb.at[0, 0]], R2.at[0], semg.at[0])

    @pl.loop(0, _CH)
    def _chunk(j):
        p = j & 1
        bp = (j // _BLK) & 1
        jj = j % _BLK

        pltpu.make_async_copy(src.at[rowb.at[bp, jj]], R2.at[p], semg.at[p]).wait()

        @plsc.parallel_loop(0, 128, step=16, unroll=2)
        def _scale(e0):
            ew16 = ewb[bp, jj, pl.ds(e0, 16)]
            for e in range(16):
                wspl = _splat(ew16, e)
                for f in range(8):
                    R2[p, e0 + e, pl.ds(f * 16, 16)] = (
                        R2[p, e0 + e, pl.ds(f * 16, 16)] * wspl)

        pltpu.async_copy(R2.at[p], sh.at[colb.at[bp, jj]], sems.at[p], add=True)

        @pl.when(j + 1 < _CH)
        def _next():
            @pl.when(j >= 1)
            def _ws():
                pltpu.make_async_copy(
                    R2.at[1 - p], sh.at[colb.at[bp, jj]], sems.at[1 - p]).wait()

            nbp = ((j + 1) // _BLK) & 1
            njj = (j + 1) % _BLK

            @pl.when(njj == 0)
            def _stage():
                nb = (j + 1) // _BLK
                pltpu.sync_copy(row3.at[w, pl.ds(nb * _BLK, _BLK)], rowb.at[nbp])
                pltpu.sync_copy(col3.at[w, pl.ds(nb * _BLK, _BLK)], colb.at[nbp])
                pltpu.sync_copy(ew3.at[w, pl.ds(nb * _BLK, _BLK)], ewb.at[nbp])

            pltpu.async_copy(src.at[rowb.at[nbp, njj]], R2.at[1 - p], semg.at[1 - p])

    pltpu.make_async_copy(R2.at[1], sh.at[colb.at[1, _BLK - 1]], sems.at[1]).wait()
    plsc.subcore_barrier()
    pltpu.sync_copy(sh.at[pl.ds(sid * 640, 640)], out_hbm.at[cid, pl.ds(sid * 640, 640)])


def _sc_scatter(src, row3, col3, ew3):
    outp = pl.kernel(
        _sc_scatter_body,
        out_type=jax.ShapeDtypeStruct((_NC, _NP, _D), jnp.float32),
        mesh=plsc.VectorSubcoreMesh(core_axis_name="c", subcore_axis_name="s"),
        scratch_types=[
            pltpu.VMEM((2, _BLK, 128), jnp.int32),
            pltpu.VMEM((2, _BLK, 128), jnp.int32),
            pltpu.VMEM((2, _BLK, 128), jnp.float32),
            pltpu.VMEM((2, 128, _D), jnp.float32),
            pltpu.VMEM_SHARED((_NP, _D), jnp.float32),
            pltpu.SemaphoreType.DMA((2,)),
            pltpu.SemaphoreType.DMA((2,)),
        ],
    )(src, row3, col3, ew3)
    return outp


_BR = 2000
_G5 = N // _BR


def _tc_pre_body(degp_ref, x_ref, xs_ref, dinv_ref):
    deg = degp_ref[:, 0:1] + degp_ref[:, 1:2] + 1.0
    dinv = jnp.where(deg > 0, lax.rsqrt(deg), 0.0)
    dinv_ref[...] = dinv
    xs_ref[...] = x_ref[...] * dinv


def _tc_pre(degp, x):
    return pl.pallas_call(
        _tc_pre_body,
        grid=(_G5,),
        in_specs=[
            pl.BlockSpec((_BR, 2), lambda i: (i, 0)),
            pl.BlockSpec((_BR, 128), lambda i: (i, 0)),
        ],
        out_specs=[
            pl.BlockSpec((_BR, 128), lambda i: (i, 0)),
            pl.BlockSpec((_BR, 1), lambda i: (i, 0)),
        ],
        out_shape=[
            jax.ShapeDtypeStruct((N, 128), jnp.float32),
            jax.ShapeDtypeStruct((N, 1), jnp.float32),
        ],
    )(degp, x)


def _tc_a_body(tp_ref, xs_ref, dinv_ref, w1_ref, b1_ref, hp_ref, st_ref):
    dinv = dinv_ref[...]
    agg = dinv * (tp_ref[0] + tp_ref[1] + xs_ref[...])
    hp = jnp.dot(agg, w1_ref[...], precision=lax.Precision.HIGHEST) + b1_ref[...]
    hp_ref[...] = hp

    @pl.when(pl.program_id(0) == 0)
    def _init():
        st_ref[...] = jnp.zeros_like(st_ref)

    st_ref[0:1, :] += jnp.sum(hp, axis=0, keepdims=True)
    st_ref[1:2, :] += jnp.sum(hp * hp, axis=0, keepdims=True)


def _tc_a(tp, xs, dinv, W1, b1):
    return pl.pallas_call(
        _tc_a_body,
        grid=(_G5,),
        in_specs=[
            pl.BlockSpec((2, _BR, 128), lambda i: (0, i, 0)),
            pl.BlockSpec((_BR, 128), lambda i: (i, 0)),
            pl.BlockSpec((_BR, 1), lambda i: (i, 0)),
            pl.BlockSpec((128, 256), lambda i: (0, 0)),
            pl.BlockSpec((1, 256), lambda i: (0, 0)),
        ],
        out_specs=[
            pl.BlockSpec((_BR, 256), lambda i: (i, 0)),
            pl.BlockSpec((2, 256), lambda i: (0, 0)),
        ],
        out_shape=[
            jax.ShapeDtypeStruct((N, 256), jnp.float32),
            jax.ShapeDtypeStruct((2, 256), jnp.float32),
        ],
    )(tp, xs, dinv, W1, b1)


def _tc_b_body(hp_ref, st_ref, g1_ref, be1_ref, w2_ref, dinv_ref, ps_ref):
    mean = st_ref[0:1, :] / N
    var = st_ref[1:2, :] / N - mean * mean
    h1 = jnp.maximum((hp_ref[...] - mean) * lax.rsqrt(var + EPS) * g1_ref[...]
                     + be1_ref[...], 0.0)
    p = jnp.dot(h1, w2_ref[...])
    ps_ref[...] = dinv_ref[...] * p


def _tc_b(hp, st, g1, be1, W2, dinv):
    return pl.pallas_call(
        _tc_b_body,
        grid=(_G5,),
        in_specs=[
            pl.BlockSpec((_BR, 256), lambda i: (i, 0)),
            pl.BlockSpec((2, 256), lambda i: (0, 0)),
            pl.BlockSpec((1, 256), lambda i: (0, 0)),
            pl.BlockSpec((1, 256), lambda i: (0, 0)),
            pl.BlockSpec((256, 128), lambda i: (0, 0)),
            pl.BlockSpec((_BR, 1), lambda i: (i, 0)),
        ],
        out_specs=pl.BlockSpec((_BR, 128), lambda i: (i, 0)),
        out_shape=jax.ShapeDtypeStruct((N, 128), jnp.float32),
    )(hp, st, g1, be1, W2, dinv)


def _tc_c_body(tp_ref, ps_ref, dinv_ref, b2_ref, ah_ref, st_ref):
    dinv = dinv_ref[...]
    ah = dinv * (tp_ref[0] + tp_ref[1] + ps_ref[...]) + b2_ref[...]
    ah_ref[...] = ah

    @pl.when(pl.program_id(0) == 0)
    def _init():
        st_ref[...] = jnp.zeros_like(st_ref)

    st_ref[0:1, :] += jnp.sum(ah, axis=0, keepdims=True)
    st_ref[1:2, :] += jnp.sum(ah * ah, axis=0, keepdims=True)


def _tc_c(tp, ps, dinv, b2):
    return pl.pallas_call(
        _tc_c_body,
        grid=(_G5,),
        in_specs=[
            pl.BlockSpec((2, _BR, 128), lambda i: (0, i, 0)),
            pl.BlockSpec((_BR, 128), lambda i: (i, 0)),
            pl.BlockSpec((_BR, 1), lambda i: (i, 0)),
            pl.BlockSpec((1, 128), lambda i: (0, 0)),
        ],
        out_specs=[
            pl.BlockSpec((_BR, 128), lambda i: (i, 0)),
            pl.BlockSpec((2, 128), lambda i: (0, 0)),
        ],
        out_shape=[
            jax.ShapeDtypeStruct((N, 128), jnp.float32),
            jax.ShapeDtypeStruct((2, 128), jnp.float32),
        ],
    )(tp, ps, dinv, b2)


def _tc_d_body(ah_ref, st_ref, g2_ref, be2_ref, dist_ref, degree_ref,
               wd_ref, bd_ref, wdeg_ref, bdeg_ref, wm_ref, bm_ref,
               wh1_ref, bh1_ref, wh2_ref, bh2_ref, emb_ref, score_ref):
    mean = st_ref[0:1, :] / N
    var = st_ref[1:2, :] / N - mean * mean
    h2 = jnp.maximum((ah_ref[...] - mean) * lax.rsqrt(var + EPS) * g2_ref[...]
                     + be2_ref[...], 0.0)
    d_feat = jnp.maximum(dist_ref[...] * wd_ref[...] + bd_ref[...], 0.0)
    deg_feat = jnp.maximum(degree_ref[...] * wdeg_ref[...] + bdeg_ref[...], 0.0)
    wm = wm_ref[...]
    emb = (jnp.dot(h2, wm[0:128]) + jnp.dot(d_feat, wm[128:256])
           + jnp.dot(deg_feat, wm[256:384]) + bm_ref[...])
    emb_ref[...] = emb
    s = jnp.maximum(jnp.dot(emb, wh1_ref[...]) + bh1_ref[...], 0.0)
    score_ref[...] = jnp.dot(s, wh2_ref[...]) + bh2_ref[...]


def _tc_d(ah, st, g2, be2, dist, degree, Wd, bd, Wdeg, bdeg, Wm, bm, Wh1, bh1, Wh2, bh2):
    full = lambda r, c: pl.BlockSpec((r, c), lambda i: (0, 0))
    return pl.pallas_call(
        _tc_d_body,
        grid=(_G5,),
        in_specs=[
            pl.BlockSpec((_BR, 128), lambda i: (i, 0)),
            full(2, 128),
            full(1, 128),
            full(1, 128),
            pl.BlockSpec((_BR, 1), lambda i: (i, 0)),
            pl.BlockSpec((_BR, 1), lambda i: (i, 0)),
            full(1, 128),
            full(1, 128),
            full(1, 128),
            full(1, 128),
            full(384, 128),
            full(1, 128),
            full(128, 128),
            full(1, 128),
            full(128, 1),
            full(1, 1),
        ],
        out_specs=[
            pl.BlockSpec((_BR, 128), lambda i: (i, 0)),
            pl.BlockSpec((_BR, 1), lambda i: (i, 0)),
        ],
        out_shape=[
            jax.ShapeDtypeStruct((N, 128), jnp.float32),
            jax.ShapeDtypeStruct((N, 1), jnp.float32),
        ],
    )(ah, st, g2, be2, dist, degree, Wd, bd, Wdeg, bdeg, Wm, bm, Wh1, bh1, Wh2, bh2)


def kernel(x, edge_index, edge_attr, dist_row_sum, degree, W1, b1, W2, b2, g1, be1, g2, be2, Wd, bd, Wdeg, bdeg, Wm, bm, Wh1, bh1, Wh2, bh2):
    ew = jnp.squeeze(edge_attr)
    row = edge_index[0]
    col = edge_index[1]

    npad = _EPAD - E
    pad_idx = (jnp.arange(npad, dtype=jnp.int32) * 37) % N
    col_p = jnp.concatenate([col, pad_idx]).reshape(_NW, _CH, 128)
    row_p = jnp.concatenate([row, pad_idx]).reshape(_NW, _CH, 128)
    ew_p = jnp.concatenate([ew, jnp.zeros((npad,), jnp.float32)]).reshape(_NW, _CH, 128)

    degp = _sc_deg(col_p, ew_p)
    xs, dinv = _tc_pre(degp.T, x)

    t1p = _sc_scatter(xs, row_p, col_p, ew_p)
    hp, st1 = _tc_a(t1p, xs, dinv, W1, b1.reshape(1, 256))
    ps = _tc_b(hp, st1, g1.reshape(1, 256), be1.reshape(1, 256), W2, dinv)

    t2p = _sc_scatter(ps, row_p, col_p, ew_p)
    ah, st2 = _tc_c(t2p, ps, dinv, b2.reshape(1, 128))
    emb, score = _tc_d(
        ah, st2, g2.reshape(1, 128), be2.reshape(1, 128), dist_row_sum, degree,
        Wd, bd.reshape(1, 128), Wdeg, bdeg.reshape(1, 128), Wm,
        bm.reshape(1, 128), Wh1, bh1.reshape(1, 128), Wh2,
        bh2.reshape(1, 1))
    return (emb, jnp.squeeze(score))


# speedup vs baseline: 1.5757x; 1.5757x over previous
import functools

import jax
import jax.numpy as jnp
from jax import lax
from jax.experimental import pallas as pl
from jax.experimental.pallas import tpu as pltpu
from jax.experimental.pallas import tpu_sc as plsc

N = 10000
E = 320000
EPS = 1e-5

_NC, _NS, _L = 2, 16, 16
_NW = _NC * _NS
_CH = 80
_EPW = _CH * 128
_EPAD = _NW * _EPW
_NP = 10240
_D = 128
_BLK = 8


def _sc_deg_body(col_hbm, ew_hbm, out_hbm, idx_v, val_v, zb, sh):
    cid = lax.axis_index("c")
    sid = lax.axis_index("s")
    w = cid * _NS + sid

    @pl.loop(0, 40)
    def _zero(i):
        zb[pl.ds(i * 16, 16)] = jnp.zeros((16,), jnp.float32)

    pltpu.sync_copy(zb, sh.at[pl.ds(sid * 640, 640)])
    pltpu.sync_copy(col_hbm.at[w], idx_v)
    pltpu.sync_copy(ew_hbm.at[w], val_v)
    plsc.subcore_barrier()

    @pl.loop(0, _CH)
    def _scat(j):
        pltpu.sync_copy(val_v.at[j], sh.at[idx_v.at[j]], add=True)

    plsc.subcore_barrier()

    @pl.when(sid == 0)
    def _out():
        pltpu.sync_copy(sh, out_hbm.at[cid])


def _sc_deg(col3, ew3):
    return pl.kernel(
        _sc_deg_body,
        out_type=jax.ShapeDtypeStruct((_NC, _NP), jnp.float32),
        mesh=plsc.VectorSubcoreMesh(core_axis_name="c", subcore_axis_name="s"),
        scratch_types=[
            pltpu.VMEM((_CH, 128), jnp.int32),
            pltpu.VMEM((_CH, 128), jnp.float32),
            pltpu.VMEM((640,), jnp.float32),
            pltpu.VMEM_SHARED((_NP,), jnp.float32),
        ],
    )(col3, ew3)


def _splat(v16, e):
    idx = jnp.full((16, 1), e, jnp.int32)
    return lax.gather(
        v16, idx,
        lax.GatherDimensionNumbers(
            offset_dims=(), collapsed_slice_dims=(0,), start_index_map=(0,)),
        slice_sizes=(1,),
        mode=lax.GatherScatterMode.PROMISE_IN_BOUNDS,
    )


def _sc_scatter_body(src, row3, col3, ew3, out_hbm, rowb, colb, ewb, R2, sh, semg, sems):
    cid = lax.axis_index("c")
    sid = lax.axis_index("s")
    w = cid * _NS + sid

    @pl.loop(0, 128)
    def _z(r):
        for f in range(8):
            R2[0, r, pl.ds(f * 16, 16)] = jnp.zeros((16,), jnp.float32)

    for k in range(5):
        pltpu.sync_copy(R2.at[0], sh.at[pl.ds(sid * 640 + k * 128, 128)])

    plsc.subcore_barrier()

    pltpu.sync_copy(row3.at[w, pl.ds(0, _BLK)], rowb.at[0])
    pltpu.sync_copy(col3.at[w, pl.ds(0, _BLK)], colb.at[0])
    pltpu.sync_copy(ew3.at[w, pl.ds(0, _BLK)], ewb.at[0])
    pltpu.async_copy(src.at[rowb.at[0, 0]], R2.at[0], semg.at[0])

    @pl.loop(0, _CH)
    def _chunk(j):
        p = j & 1
        bp = (j // _BLK) & 1
        jj = j % _BLK

        pltpu.make_async_copy(src.at[rowb.at[bp, jj]], R2.at[p], semg.at[p]).wait()

        @plsc.parallel_loop(0, 128, step=16, unroll=8)
        def _scale(e0):
            ew16 = ewb[bp, jj, pl.ds(e0, 16)]
            for e in range(16):
                wspl = _splat(ew16, e)
                for f in range(8):
                    R2[p, e0 + e, pl.ds(f * 16, 16)] = (
                        R2[p, e0 + e, pl.ds(f * 16, 16)] * wspl)

        pltpu.async_copy(R2.at[p], sh.at[colb.at[bp, jj]], sems.at[p], add=True)

        @pl.when(j + 1 < _CH)
        def _next():
            @pl.when(j >= 1)
            def _ws():
                pltpu.make_async_copy(
                    R2.at[1 - p], sh.at[colb.at[bp, jj]], sems.at[1 - p]).wait()

            nbp = ((j + 1) // _BLK) & 1
            njj = (j + 1) % _BLK

            @pl.when(njj == 0)
            def _stage():
                nb = (j + 1) // _BLK
                pltpu.sync_copy(row3.at[w, pl.ds(nb * _BLK, _BLK)], rowb.at[nbp])
                pltpu.sync_copy(col3.at[w, pl.ds(nb * _BLK, _BLK)], colb.at[nbp])
                pltpu.sync_copy(ew3.at[w, pl.ds(nb * _BLK, _BLK)], ewb.at[nbp])

            pltpu.async_copy(src.at[rowb.at[nbp, njj]], R2.at[1 - p], semg.at[1 - p])

    pltpu.make_async_copy(R2.at[1], sh.at[colb.at[1, _BLK - 1]], sems.at[1]).wait()
    plsc.subcore_barrier()
    pltpu.sync_copy(sh.at[pl.ds(sid * 640, 640)], out_hbm.at[cid, pl.ds(sid * 640, 640)])


def _sc_scatter(src, row3, col3, ew3):
    outp = pl.kernel(
        _sc_scatter_body,
        out_type=jax.ShapeDtypeStruct((_NC, _NP, _D), jnp.float32),
        mesh=plsc.VectorSubcoreMesh(core_axis_name="c", subcore_axis_name="s"),
        scratch_types=[
            pltpu.VMEM((2, _BLK, 128), jnp.int32),
            pltpu.VMEM((2, _BLK, 128), jnp.int32),
            pltpu.VMEM((2, _BLK, 128), jnp.float32),
            pltpu.VMEM((2, 128, _D), jnp.float32),
            pltpu.VMEM_SHARED((_NP, _D), jnp.float32),
            pltpu.SemaphoreType.DMA((2,)),
            pltpu.SemaphoreType.DMA((2,)),
        ],
    )(src, row3, col3, ew3)
    return outp


_BR = 2000
_G5 = N // _BR


def _tc_pre_body(degp_ref, x_ref, xs_ref, dinv_ref):
    deg = degp_ref[:, 0:1] + degp_ref[:, 1:2] + 1.0
    dinv = jnp.where(deg > 0, lax.rsqrt(deg), 0.0)
    dinv_ref[...] = dinv
    xs_ref[...] = x_ref[...] * dinv


def _tc_pre(degp, x):
    return pl.pallas_call(
        _tc_pre_body,
        grid=(_G5,),
        in_specs=[
            pl.BlockSpec((_BR, 2), lambda i: (i, 0)),
            pl.BlockSpec((_BR, 128), lambda i: (i, 0)),
        ],
        out_specs=[
            pl.BlockSpec((_BR, 128), lambda i: (i, 0)),
            pl.BlockSpec((_BR, 1), lambda i: (i, 0)),
        ],
        out_shape=[
            jax.ShapeDtypeStruct((N, 128), jnp.float32),
            jax.ShapeDtypeStruct((N, 1), jnp.float32),
        ],
    )(degp, x)


def _tc_a_body(tp_ref, xs_ref, dinv_ref, w1_ref, b1_ref, hp_ref, st_ref):
    dinv = dinv_ref[...]
    agg = dinv * (tp_ref[0] + tp_ref[1] + xs_ref[...])
    hp = jnp.dot(agg, w1_ref[...], precision=lax.Precision.HIGHEST) + b1_ref[...]
    hp_ref[...] = hp

    @pl.when(pl.program_id(0) == 0)
    def _init():
        st_ref[...] = jnp.zeros_like(st_ref)

    st_ref[0:1, :] += jnp.sum(hp, axis=0, keepdims=True)
    st_ref[1:2, :] += jnp.sum(hp * hp, axis=0, keepdims=True)


def _tc_a(tp, xs, dinv, W1, b1):
    return pl.pallas_call(
        _tc_a_body,
        grid=(_G5,),
        in_specs=[
            pl.BlockSpec((2, _BR, 128), lambda i: (0, i, 0)),
            pl.BlockSpec((_BR, 128), lambda i: (i, 0)),
            pl.BlockSpec((_BR, 1), lambda i: (i, 0)),
            pl.BlockSpec((128, 256), lambda i: (0, 0)),
            pl.BlockSpec((1, 256), lambda i: (0, 0)),
        ],
        out_specs=[
            pl.BlockSpec((_BR, 256), lambda i: (i, 0)),
            pl.BlockSpec((2, 256), lambda i: (0, 0)),
        ],
        out_shape=[
            jax.ShapeDtypeStruct((N, 256), jnp.float32),
            jax.ShapeDtypeStruct((2, 256), jnp.float32),
        ],
    )(tp, xs, dinv, W1, b1)


def _tc_b_body(hp_ref, st_ref, g1_ref, be1_ref, w2_ref, dinv_ref, ps_ref):
    mean = st_ref[0:1, :] / N
    var = st_ref[1:2, :] / N - mean * mean
    h1 = jnp.maximum((hp_ref[...] - mean) * lax.rsqrt(var + EPS) * g1_ref[...]
                     + be1_ref[...], 0.0)
    p = jnp.dot(h1, w2_ref[...])
    ps_ref[...] = dinv_ref[...] * p


def _tc_b(hp, st, g1, be1, W2, dinv):
    return pl.pallas_call(
        _tc_b_body,
        grid=(_G5,),
        in_specs=[
            pl.BlockSpec((_BR, 256), lambda i: (i, 0)),
            pl.BlockSpec((2, 256), lambda i: (0, 0)),
            pl.BlockSpec((1, 256), lambda i: (0, 0)),
            pl.BlockSpec((1, 256), lambda i: (0, 0)),
            pl.BlockSpec((256, 128), lambda i: (0, 0)),
            pl.BlockSpec((_BR, 1), lambda i: (i, 0)),
        ],
        out_specs=pl.BlockSpec((_BR, 128), lambda i: (i, 0)),
        out_shape=jax.ShapeDtypeStruct((N, 128), jnp.float32),
    )(hp, st, g1, be1, W2, dinv)


def _tc_c_body(tp_ref, ps_ref, dinv_ref, b2_ref, ah_ref, st_ref):
    dinv = dinv_ref[...]
    ah = dinv * (tp_ref[0] + tp_ref[1] + ps_ref[...]) + b2_ref[...]
    ah_ref[...] = ah

    @pl.when(pl.program_id(0) == 0)
    def _init():
        st_ref[...] = jnp.zeros_like(st_ref)

    st_ref[0:1, :] += jnp.sum(ah, axis=0, keepdims=True)
    st_ref[1:2, :] += jnp.sum(ah * ah, axis=0, keepdims=True)


def _tc_c(tp, ps, dinv, b2):
    return pl.pallas_call(
        _tc_c_body,
        grid=(_G5,),
        in_specs=[
            pl.BlockSpec((2, _BR, 128), lambda i: (0, i, 0)),
            pl.BlockSpec((_BR, 128), lambda i: (i, 0)),
            pl.BlockSpec((_BR, 1), lambda i: (i, 0)),
            pl.BlockSpec((1, 128), lambda i: (0, 0)),
        ],
        out_specs=[
            pl.BlockSpec((_BR, 128), lambda i: (i, 0)),
            pl.BlockSpec((2, 128), lambda i: (0, 0)),
        ],
        out_shape=[
            jax.ShapeDtypeStruct((N, 128), jnp.float32),
            jax.ShapeDtypeStruct((2, 128), jnp.float32),
        ],
    )(tp, ps, dinv, b2)


def _tc_d_body(ah_ref, st_ref, g2_ref, be2_ref, dist_ref, degree_ref,
               wd_ref, bd_ref, wdeg_ref, bdeg_ref, wm_ref, bm_ref,
               wh1_ref, bh1_ref, wh2_ref, bh2_ref, emb_ref, score_ref):
    mean = st_ref[0:1, :] / N
    var = st_ref[1:2, :] / N - mean * mean
    h2 = jnp.maximum((ah_ref[...] - mean) * lax.rsqrt(var + EPS) * g2_ref[...]
                     + be2_ref[...], 0.0)
    d_feat = jnp.maximum(dist_ref[...] * wd_ref[...] + bd_ref[...], 0.0)
    deg_feat = jnp.maximum(degree_ref[...] * wdeg_ref[...] + bdeg_ref[...], 0.0)
    wm = wm_ref[...]
    emb = (jnp.dot(h2, wm[0:128]) + jnp.dot(d_feat, wm[128:256])
           + jnp.dot(deg_feat, wm[256:384]) + bm_ref[...])
    emb_ref[...] = emb
    s = jnp.maximum(jnp.dot(emb, wh1_ref[...]) + bh1_ref[...], 0.0)
    score_ref[...] = jnp.dot(s, wh2_ref[...]) + bh2_ref[...]


def _tc_d(ah, st, g2, be2, dist, degree, Wd, bd, Wdeg, bdeg, Wm, bm, Wh1, bh1, Wh2, bh2):
    full = lambda r, c: pl.BlockSpec((r, c), lambda i: (0, 0))
    return pl.pallas_call(
        _tc_d_body,
        grid=(_G5,),
        in_specs=[
            pl.BlockSpec((_BR, 128), lambda i: (i, 0)),
            full(2, 128),
            full(1, 128),
            full(1, 128),
            pl.BlockSpec((_BR, 1), lambda i: (i, 0)),
            pl.BlockSpec((_BR, 1), lambda i: (i, 0)),
            full(1, 128),
            full(1, 128),
            full(1, 128),
            full(1, 128),
            full(384, 128),
            full(1, 128),
            full(128, 128),
            full(1, 128),
            full(128, 1),
            full(1, 1),
        ],
        out_specs=[
            pl.BlockSpec((_BR, 128), lambda i: (i, 0)),
            pl.BlockSpec((_BR, 1), lambda i: (i, 0)),
        ],
        out_shape=[
            jax.ShapeDtypeStruct((N, 128), jnp.float32),
            jax.ShapeDtypeStruct((N, 1), jnp.float32),
        ],
    )(ah, st, g2, be2, dist, degree, Wd, bd, Wdeg, bdeg, Wm, bm, Wh1, bh1, Wh2, bh2)


def kernel(x, edge_index, edge_attr, dist_row_sum, degree, W1, b1, W2, b2, g1, be1, g2, be2, Wd, bd, Wdeg, bdeg, Wm, bm, Wh1, bh1, Wh2, bh2):
    ew = jnp.squeeze(edge_attr)
    row = edge_index[0]
    col = edge_index[1]

    npad = _EPAD - E
    pad_idx = (jnp.arange(npad, dtype=jnp.int32) * 37) % N
    col_p = jnp.concatenate([col, pad_idx]).reshape(_NW, _CH, 128)
    row_p = jnp.concatenate([row, pad_idx]).reshape(_NW, _CH, 128)
    ew_p = jnp.concatenate([ew, jnp.zeros((npad,), jnp.float32)]).reshape(_NW, _CH, 128)

    degp = _sc_deg(col_p, ew_p)
    xs, dinv = _tc_pre(degp.T, x)

    t1p = _sc_scatter(xs, row_p, col_p, ew_p)
    hp, st1 = _tc_a(t1p, xs, dinv, W1, b1.reshape(1, 256))
    ps = _tc_b(hp, st1, g1.reshape(1, 256), be1.reshape(1, 256), W2, dinv)

    t2p = _sc_scatter(ps, row_p, col_p, ew_p)
    ah, st2 = _tc_c(t2p, ps, dinv, b2.reshape(1, 128))
    emb, score = _tc_d(
        ah, st2, g2.reshape(1, 128), be2.reshape(1, 128), dist_row_sum, degree,
        Wd, bd.reshape(1, 128), Wdeg, bdeg.reshape(1, 128), Wm,
        bm.reshape(1, 128), Wh1, bh1.reshape(1, 128), Wh2,
        bh2.reshape(1, 1))
    return (emb, jnp.squeeze(score))


# speedup vs baseline: 1.9158x; 1.2158x over previous
import functools

import jax
import jax.numpy as jnp
from jax import lax
from jax.experimental import pallas as pl
from jax.experimental.pallas import tpu as pltpu
from jax.experimental.pallas import tpu_sc as plsc

N = 10000
E = 320000
EPS = 1e-5

_NC, _NS, _L = 2, 16, 16
_NW = _NC * _NS
_CH = 80
_EPW = _CH * 128
_EPAD = _NW * _EPW
_NP = 10240
_D = 128
_BLK = 8


def _sc_deg_body(col_hbm, ew_hbm, out_hbm, idx_v, val_v, zb, sh):
    cid = lax.axis_index("c")
    sid = lax.axis_index("s")
    w = cid * _NS + sid

    @pl.loop(0, 40)
    def _zero(i):
        zb[pl.ds(i * 16, 16)] = jnp.zeros((16,), jnp.float32)

    pltpu.sync_copy(zb, sh.at[pl.ds(sid * 640, 640)])
    pltpu.sync_copy(col_hbm.at[w], idx_v)
    pltpu.sync_copy(ew_hbm.at[w], val_v)
    plsc.subcore_barrier()

    @pl.loop(0, _CH)
    def _scat(j):
        pltpu.sync_copy(val_v.at[j], sh.at[idx_v.at[j]], add=True)

    plsc.subcore_barrier()

    @pl.when(sid == 0)
    def _out():
        pltpu.sync_copy(sh, out_hbm.at[cid])


def _sc_deg(col3, ew3):
    return pl.kernel(
        _sc_deg_body,
        out_type=jax.ShapeDtypeStruct((_NC, _NP), jnp.float32),
        mesh=plsc.VectorSubcoreMesh(core_axis_name="c", subcore_axis_name="s"),
        scratch_types=[
            pltpu.VMEM((_CH, 128), jnp.int32),
            pltpu.VMEM((_CH, 128), jnp.float32),
            pltpu.VMEM((640,), jnp.float32),
            pltpu.VMEM_SHARED((_NP,), jnp.float32),
        ],
    )(col3, ew3)


def _splat(v16, e):
    idx = jnp.full((16, 1), e, jnp.int32)
    return lax.gather(
        v16, idx,
        lax.GatherDimensionNumbers(
            offset_dims=(), collapsed_slice_dims=(0,), start_index_map=(0,)),
        slice_sizes=(1,),
        mode=lax.GatherScatterMode.PROMISE_IN_BOUNDS,
    )


_CE = 64
_CH2 = _EPW // _CE
_BLK2 = 16


def _sc_scatter_body(src, row3, col3, ew3, out_hbm, rowb, colb, ewb, R3, sh, semg, sems):
    cid = lax.axis_index("c")
    sid = lax.axis_index("s")
    w = cid * _NS + sid

    @pl.loop(0, _CE)
    def _z(r):
        for f in range(8):
            R3[0, r, pl.ds(f * 16, 16)] = jnp.zeros((16,), jnp.float32)

    for k in range(10):
        pltpu.sync_copy(R3.at[0], sh.at[pl.ds(sid * 640 + k * _CE, _CE)])

    plsc.subcore_barrier()

    pltpu.sync_copy(row3.at[w, pl.ds(0, _BLK2)], rowb.at[0])
    pltpu.sync_copy(col3.at[w, pl.ds(0, _BLK2)], colb.at[0])
    pltpu.sync_copy(ew3.at[w, pl.ds(0, _BLK2)], ewb.at[0])
    pltpu.async_copy(src.at[rowb.at[0, 0]], R3.at[0], semg.at[0])
    pltpu.async_copy(src.at[rowb.at[0, 1]], R3.at[1], semg.at[1])

    @pl.loop(0, _CH2)
    def _chunk(j):
        p = lax.rem(j, 3)
        bp = (j // _BLK2) & 1
        jj = j % _BLK2

        pltpu.make_async_copy(src.at[rowb.at[bp, jj]], R3.at[p], semg.at[p]).wait()

        @plsc.parallel_loop(0, _CE, step=16, unroll=4)
        def _scale(e0):
            ew16 = ewb[bp, jj, pl.ds(e0, 16)]
            for e in range(16):
                wspl = _splat(ew16, e)
                for f in range(8):
                    R3[p, e0 + e, pl.ds(f * 16, 16)] = (
                        R3[p, e0 + e, pl.ds(f * 16, 16)] * wspl)

        pltpu.async_copy(R3.at[p], sh.at[colb.at[bp, jj]], sems.at[p], add=True)

        @pl.when(j + 2 < _CH2)
        def _next():
            p2 = lax.rem(j + 2, 3)

            @pl.when(j >= 1)
            def _ws():
                pltpu.make_async_copy(
                    R3.at[p2], sh.at[colb.at[bp, jj]], sems.at[p2]).wait()

            nbp = ((j + 2) // _BLK2) & 1
            njj = (j + 2) % _BLK2

            @pl.when(njj == 0)
            def _stage():
                nb = (j + 2) // _BLK2
                pltpu.sync_copy(row3.at[w, pl.ds(nb * _BLK2, _BLK2)], rowb.at[nbp])
                pltpu.sync_copy(col3.at[w, pl.ds(nb * _BLK2, _BLK2)], colb.at[nbp])
                pltpu.sync_copy(ew3.at[w, pl.ds(nb * _BLK2, _BLK2)], ewb.at[nbp])

            pltpu.async_copy(src.at[rowb.at[nbp, njj]], R3.at[p2], semg.at[p2])

    pltpu.make_async_copy(R3.at[1], sh.at[colb.at[1, 0]], sems.at[1]).wait()
    pltpu.make_async_copy(R3.at[2], sh.at[colb.at[1, 0]], sems.at[2]).wait()
    pltpu.make_async_copy(R3.at[0], sh.at[colb.at[1, 0]], sems.at[0]).wait()
    plsc.subcore_barrier()
    pltpu.sync_copy(sh.at[pl.ds(sid * 640, 640)], out_hbm.at[cid, pl.ds(sid * 640, 640)])


def _sc_scatter(src, row3, col3, ew3):
    outp = pl.kernel(
        _sc_scatter_body,
        out_type=jax.ShapeDtypeStruct((_NC, _NP, _D), jnp.float32),
        mesh=plsc.VectorSubcoreMesh(core_axis_name="c", subcore_axis_name="s"),
        scratch_types=[
            pltpu.VMEM((2, _BLK2, _CE), jnp.int32),
            pltpu.VMEM((2, _BLK2, _CE), jnp.int32),
            pltpu.VMEM((2, _BLK2, _CE), jnp.float32),
            pltpu.VMEM((3, _CE, _D), jnp.float32),
            pltpu.VMEM_SHARED((_NP, _D), jnp.float32),
            pltpu.SemaphoreType.DMA((3,)),
            pltpu.SemaphoreType.DMA((3,)),
        ],
    )(src, row3, col3, ew3)
    return outp


_BR = 2000
_G5 = N // _BR


def _tc_pre_body(degp_ref, x_ref, xs_ref, dinv_ref):
    deg = degp_ref[:, 0:1] + degp_ref[:, 1:2] + 1.0
    dinv = jnp.where(deg > 0, lax.rsqrt(deg), 0.0)
    dinv_ref[...] = dinv
    xs_ref[...] = x_ref[...] * dinv


def _tc_pre(degp, x):
    return pl.pallas_call(
        _tc_pre_body,
        grid=(_G5,),
        in_specs=[
            pl.BlockSpec((_BR, 2), lambda i: (i, 0)),
            pl.BlockSpec((_BR, 128), lambda i: (i, 0)),
        ],
        out_specs=[
            pl.BlockSpec((_BR, 128), lambda i: (i, 0)),
            pl.BlockSpec((_BR, 1), lambda i: (i, 0)),
        ],
        out_shape=[
            jax.ShapeDtypeStruct((N, 128), jnp.float32),
            jax.ShapeDtypeStruct((N, 1), jnp.float32),
        ],
    )(degp, x)


def _tc_a_body(tp_ref, xs_ref, dinv_ref, w1_ref, b1_ref, hp_ref, st_ref):
    dinv = dinv_ref[...]
    agg = dinv * (tp_ref[0] + tp_ref[1] + xs_ref[...])
    hp = jnp.dot(agg, w1_ref[...], precision=lax.Precision.HIGHEST) + b1_ref[...]
    hp_ref[...] = hp

    @pl.when(pl.program_id(0) == 0)
    def _init():
        st_ref[...] = jnp.zeros_like(st_ref)

    st_ref[0:1, :] += jnp.sum(hp, axis=0, keepdims=True)
    st_ref[1:2, :] += jnp.sum(hp * hp, axis=0, keepdims=True)


def _tc_a(tp, xs, dinv, W1, b1):
    return pl.pallas_call(
        _tc_a_body,
        grid=(_G5,),
        in_specs=[
            pl.BlockSpec((2, _BR, 128), lambda i: (0, i, 0)),
            pl.BlockSpec((_BR, 128), lambda i: (i, 0)),
            pl.BlockSpec((_BR, 1), lambda i: (i, 0)),
            pl.BlockSpec((128, 256), lambda i: (0, 0)),
            pl.BlockSpec((1, 256), lambda i: (0, 0)),
        ],
        out_specs=[
            pl.BlockSpec((_BR, 256), lambda i: (i, 0)),
            pl.BlockSpec((2, 256), lambda i: (0, 0)),
        ],
        out_shape=[
            jax.ShapeDtypeStruct((N, 256), jnp.float32),
            jax.ShapeDtypeStruct((2, 256), jnp.float32),
        ],
    )(tp, xs, dinv, W1, b1)


def _tc_b_body(hp_ref, st_ref, g1_ref, be1_ref, w2_ref, dinv_ref, ps_ref):
    mean = st_ref[0:1, :] / N
    var = st_ref[1:2, :] / N - mean * mean
    h1 = jnp.maximum((hp_ref[...] - mean) * lax.rsqrt(var + EPS) * g1_ref[...]
                     + be1_ref[...], 0.0)
    p = jnp.dot(h1, w2_ref[...])
    ps_ref[...] = dinv_ref[...] * p


def _tc_b(hp, st, g1, be1, W2, dinv):
    return pl.pallas_call(
        _tc_b_body,
        grid=(_G5,),
        in_specs=[
            pl.BlockSpec((_BR, 256), lambda i: (i, 0)),
            pl.BlockSpec((2, 256), lambda i: (0, 0)),
            pl.BlockSpec((1, 256), lambda i: (0, 0)),
            pl.BlockSpec((1, 256), lambda i: (0, 0)),
            pl.BlockSpec((256, 128), lambda i: (0, 0)),
            pl.BlockSpec((_BR, 1), lambda i: (i, 0)),
        ],
        out_specs=pl.BlockSpec((_BR, 128), lambda i: (i, 0)),
        out_shape=jax.ShapeDtypeStruct((N, 128), jnp.float32),
    )(hp, st, g1, be1, W2, dinv)


def _tc_c_body(tp_ref, ps_ref, dinv_ref, b2_ref, ah_ref, st_ref):
    dinv = dinv_ref[...]
    ah = dinv * (tp_ref[0] + tp_ref[1] + ps_ref[...]) + b2_ref[...]
    ah_ref[...] = ah

    @pl.when(pl.program_id(0) == 0)
    def _init():
        st_ref[...] = jnp.zeros_like(st_ref)

    st_ref[0:1, :] += jnp.sum(ah, axis=0, keepdims=True)
    st_ref[1:2, :] += jnp.sum(ah * ah, axis=0, keepdims=True)


def _tc_c(tp, ps, dinv, b2):
    return pl.pallas_call(
        _tc_c_body,
        grid=(_G5,),
        in_specs=[
            pl.BlockSpec((2, _BR, 128), lambda i: (0, i, 0)),
            pl.BlockSpec((_BR, 128), lambda i: (i, 0)),
            pl.BlockSpec((_BR, 1), lambda i: (i, 0)),
            pl.BlockSpec((1, 128), lambda i: (0, 0)),
        ],
        out_specs=[
            pl.BlockSpec((_BR, 128), lambda i: (i, 0)),
            pl.BlockSpec((2, 128), lambda i: (0, 0)),
        ],
        out_shape=[
            jax.ShapeDtypeStruct((N, 128), jnp.float32),
            jax.ShapeDtypeStruct((2, 128), jnp.float32),
        ],
    )(tp, ps, dinv, b2)


def _tc_d_body(ah_ref, st_ref, g2_ref, be2_ref, dist_ref, degree_ref,
               wd_ref, bd_ref, wdeg_ref, bdeg_ref, wm_ref, bm_ref,
               wh1_ref, bh1_ref, wh2_ref, bh2_ref, emb_ref, score_ref):
    mean = st_ref[0:1, :] / N
    var = st_ref[1:2, :] / N - mean * mean
    h2 = jnp.maximum((ah_ref[...] - mean) * lax.rsqrt(var + EPS) * g2_ref[...]
                     + be2_ref[...], 0.0)
    d_feat = jnp.maximum(dist_ref[...] * wd_ref[...] + bd_ref[...], 0.0)
    deg_feat = jnp.maximum(degree_ref[...] * wdeg_ref[...] + bdeg_ref[...], 0.0)
    wm = wm_ref[...]
    emb = (jnp.dot(h2, wm[0:128]) + jnp.dot(d_feat, wm[128:256])
           + jnp.dot(deg_feat, wm[256:384]) + bm_ref[...])
    emb_ref[...] = emb
    s = jnp.maximum(jnp.dot(emb, wh1_ref[...]) + bh1_ref[...], 0.0)
    score_ref[...] = jnp.dot(s, wh2_ref[...]) + bh2_ref[...]


def _tc_d(ah, st, g2, be2, dist, degree, Wd, bd, Wdeg, bdeg, Wm, bm, Wh1, bh1, Wh2, bh2):
    full = lambda r, c: pl.BlockSpec((r, c), lambda i: (0, 0))
    return pl.pallas_call(
        _tc_d_body,
        grid=(_G5,),
        in_specs=[
            pl.BlockSpec((_BR, 128), lambda i: (i, 0)),
            full(2, 128),
            full(1, 128),
            full(1, 128),
            pl.BlockSpec((_BR, 1), lambda i: (i, 0)),
            pl.BlockSpec((_BR, 1), lambda i: (i, 0)),
            full(1, 128),
            full(1, 128),
            full(1, 128),
            full(1, 128),
            full(384, 128),
            full(1, 128),
            full(128, 128),
            full(1, 128),
            full(128, 1),
            full(1, 1),
        ],
        out_specs=[
            pl.BlockSpec((_BR, 128), lambda i: (i, 0)),
            pl.BlockSpec((_BR, 1), lambda i: (i, 0)),
        ],
        out_shape=[
            jax.ShapeDtypeStruct((N, 128), jnp.float32),
            jax.ShapeDtypeStruct((N, 1), jnp.float32),
        ],
    )(ah, st, g2, be2, dist, degree, Wd, bd, Wdeg, bdeg, Wm, bm, Wh1, bh1, Wh2, bh2)


def kernel(x, edge_index, edge_attr, dist_row_sum, degree, W1, b1, W2, b2, g1, be1, g2, be2, Wd, bd, Wdeg, bdeg, Wm, bm, Wh1, bh1, Wh2, bh2):
    ew = jnp.squeeze(edge_attr)
    row = edge_index[0]
    col = edge_index[1]

    npad = _EPAD - E
    pad_idx = (jnp.arange(npad, dtype=jnp.int32) * 37) % N
    col_f = jnp.concatenate([col, pad_idx])
    row_f = jnp.concatenate([row, pad_idx])
    ew_f = jnp.concatenate([ew, jnp.zeros((npad,), jnp.float32)])
    col_p = col_f.reshape(_NW, _CH, 128)
    ew_p = ew_f.reshape(_NW, _CH, 128)
    col_p2 = col_f.reshape(_NW, _CH2, _CE)
    row_p2 = row_f.reshape(_NW, _CH2, _CE)
    ew_p2 = ew_f.reshape(_NW, _CH2, _CE)

    degp = _sc_deg(col_p, ew_p)
    xs, dinv = _tc_pre(degp.T, x)

    t1p = _sc_scatter(xs, row_p2, col_p2, ew_p2)
    hp, st1 = _tc_a(t1p, xs, dinv, W1, b1.reshape(1, 256))
    ps = _tc_b(hp, st1, g1.reshape(1, 256), be1.reshape(1, 256), W2, dinv)

    t2p = _sc_scatter(ps, row_p2, col_p2, ew_p2)
    ah, st2 = _tc_c(t2p, ps, dinv, b2.reshape(1, 128))
    emb, score = _tc_d(
        ah, st2, g2.reshape(1, 128), be2.reshape(1, 128), dist_row_sum, degree,
        Wd, bd.reshape(1, 128), Wdeg, bdeg.reshape(1, 128), Wm,
        bm.reshape(1, 128), Wh1, bh1.reshape(1, 128), Wh2,
        bh2.reshape(1, 1))
    return (emb, jnp.squeeze(score))
